# baseline (speedup 1.0000x reference)
import functools

import jax
import jax.numpy as jnp
from jax import lax
from jax.experimental import pallas as pl
from jax.experimental.pallas import tpu as pltpu
from jax.experimental.pallas import tpu_sc as plsc

DIM = 64
MARGIN = 6.0
NC, NS = 2, 16
NW = NC * NS
IW = 128
CH = 8
PB = 1000


def _tc_prep(ent_re, ent_im, rel_ph):
    n = ent_re.shape[0]

    def body(re_ref, im_ref, ph_ref, e2_ref, r2_ref):
        e2_ref[...] = jnp.concatenate([re_ref[...], im_ref[...]], axis=1)
        p = ph_ref[...]
        r2_ref[...] = jnp.concatenate([jnp.cos(p), jnp.sin(p)], axis=1)

    in_spec = pl.BlockSpec((PB, DIM), lambda i: (i, 0))
    out_spec = pl.BlockSpec((PB, 2 * DIM), lambda i: (i, 0))
    t = jax.ShapeDtypeStruct((n, 2 * DIM), jnp.float32)
    return pl.pallas_call(
        body,
        grid=(n // PB,),
        in_specs=[in_spec] * 3,
        out_specs=[out_spec] * 2,
        out_shape=[t, t],
        compiler_params=pltpu.CompilerParams(
            dimension_semantics=("parallel",)),
    )(ent_re, ent_im, rel_ph)


def _sc_gather(heads, rels, tails, e2, r2, total):
    b_per_w = CH * IW
    mesh = plsc.VectorSubcoreMesh(core_axis_name="c", subcore_axis_name="s")
    row_t = jax.ShapeDtypeStruct((total, 2 * DIM), jnp.float32)

    @functools.partial(
        pl.kernel, mesh=mesh,
        out_type=[row_t] * 3,
        scratch_types=[pltpu.VMEM((CH, IW), jnp.int32)] * 3
        + [pltpu.VMEM((2 * IW, 2 * DIM), jnp.float32)] * 2
        + [pltpu.SemaphoreType.DMA] * 2,
    )
    def k(h_hbm, r_hbm, t_hbm, e2_hbm, r2_hbm,
          o_h, o_t, o_r,
          hidx, ridx, tidx, buf0, buf1, sem_g, sem_w):
        wid = lax.axis_index("s") * NC + lax.axis_index("c")
        base = wid * b_per_w
        pltpu.sync_copy(h_hbm.at[wid], hidx)
        pltpu.sync_copy(t_hbm.at[wid], tidx)
        pltpu.sync_copy(r_hbm.at[wid], ridx)

        bufs = (buf0, buf1)
        writes = [None, None]
        step = 0
        for idx_v, table, out in ((hidx, e2_hbm, o_h),
                                  (tidx, e2_hbm, o_t),
                                  (ridx, r2_hbm, o_r)):
            for h in range(CH // 2):
                b = step % 2
                if writes[b] is not None:
                    writes[b].wait()
                g0 = pltpu.async_copy(table.at[idx_v.at[2 * h]],
                                      bufs[b].at[pl.ds(0, IW)], sem_g)
                g1 = pltpu.async_copy(table.at[idx_v.at[2 * h + 1]],
                                      bufs[b].at[pl.ds(IW, IW)], sem_g)
                g0.wait()
                g1.wait()
                writes[b] = pltpu.async_copy(
                    bufs[b], out.at[pl.ds(base + h * 2 * IW, 2 * IW)], sem_w)
                step += 1
        writes[0].wait()
        writes[1].wait()

    return k(heads, rels, tails, e2, r2)


def _tc_loss(g_h, g_t, g_r, batch):
    w = 2048
    g = batch // w

    def body(gh_p, gt_p, gr_p, gh_n, gt_n, gr_n, out):
        def mag(gh, gt, gr):
            hre, him = gh[:, :DIM], gh[:, DIM:]
            tre, tim = gt[:, :DIM], gt[:, DIM:]
            c, s = gr[:, :DIM], gr[:, DIM:]
            dre = hre * c - him * s - tre
            dim = hre * s + him * c - tim
            return jnp.sum(jnp.sqrt(dre * dre + dim * dim + 1e-9), axis=-1)

        ms = jnp.maximum(MARGIN - mag(gh_n[...], gt_n[...], gr_n[...])
                         + mag(gh_p[...], gt_p[...], gr_p[...]), 0.0)
        i = pl.program_id(0)

        @pl.when(i == 0)
        def _():
            out[...] = jnp.zeros((1, 1), jnp.float32)

        out[...] += jnp.sum(ms).reshape(1, 1)

        @pl.when(i == g - 1)
        def _():
            out[...] = out[...] / batch

    pos_spec = pl.BlockSpec((w, 2 * DIM), lambda i: (i, 0))
    neg_spec = pl.BlockSpec((w, 2 * DIM), lambda i: (i + g, 0))
    out = pl.pallas_call(
        body,
        grid=(g,),
        in_specs=[pos_spec] * 3 + [neg_spec] * 3,
        out_specs=pl.BlockSpec((1, 1), lambda i: (0, 0)),
        out_shape=jax.ShapeDtypeStruct((1, 1), jnp.float32),
    )(g_h, g_t, g_r, g_h, g_t, g_r)
    return out[0, 0]


def kernel(positive_triples, negative_triples, entity_re, entity_im,
           relation_phase):
    batch = positive_triples.shape[0]
    total = 2 * batch
    pt = positive_triples.astype(jnp.int32)
    nt = negative_triples.astype(jnp.int32)
    heads = jnp.concatenate([pt[:, 0], nt[:, 0]]).reshape(NW, CH, IW)
    rels = jnp.concatenate([pt[:, 1], nt[:, 1]]).reshape(NW, CH, IW)
    tails = jnp.concatenate([pt[:, 2], nt[:, 2]]).reshape(NW, CH, IW)
    e2, r2 = _tc_prep(entity_re, entity_im, relation_phase)
    g_h, g_t, g_r = _sc_gather(heads, rels, tails, e2, r2, total)
    return _tc_loss(g_h, g_t, g_r, batch)


# speedup vs baseline: 1.0434x; 1.0434x over previous
import functools

import jax
import jax.numpy as jnp
from jax import lax
from jax.experimental import pallas as pl
from jax.experimental.pallas import tpu as pltpu
from jax.experimental.pallas import tpu_sc as plsc

DIM = 64
MARGIN = 6.0
NC, NS = 2, 16
NW = NC * NS
IW = 128
CH = 8
PB = 1000


def _tc_prep(ent_re, ent_im, rel_ph):
    n = ent_re.shape[0]

    def body(re_ref, im_ref, ph_ref, e2_ref, r2_ref):
        e2_ref[...] = jnp.concatenate([re_ref[...], im_ref[...]], axis=1)
        p = ph_ref[...]
        r2_ref[...] = jnp.concatenate([p, p], axis=1)

    in_spec = pl.BlockSpec((PB, DIM), lambda i: (i, 0))
    out_spec = pl.BlockSpec((PB, 2 * DIM), lambda i: (i, 0))
    t = jax.ShapeDtypeStruct((n, 2 * DIM), jnp.float32)
    return pl.pallas_call(
        body,
        grid=(n // PB,),
        in_specs=[in_spec] * 3,
        out_specs=[out_spec] * 2,
        out_shape=[t, t],
        compiler_params=pltpu.CompilerParams(
            dimension_semantics=("parallel",)),
    )(ent_re, ent_im, rel_ph)


def _sc_gather(heads, rels, tails, e2, r2, total):
    b_per_w = CH * IW
    mesh = plsc.VectorSubcoreMesh(core_axis_name="c", subcore_axis_name="s")
    row_t = jax.ShapeDtypeStruct((total, 2 * DIM), jnp.float32)

    @functools.partial(
        pl.kernel, mesh=mesh,
        out_type=[row_t] * 3,
        scratch_types=[pltpu.VMEM((CH, IW), jnp.int32)] * 3
        + [pltpu.VMEM((2 * IW, 2 * DIM), jnp.float32)] * 2
        + [pltpu.SemaphoreType.DMA] * 2,
    )
    def k(h_hbm, r_hbm, t_hbm, e2_hbm, r2_hbm,
          o_h, o_t, o_r,
          hidx, ridx, tidx, buf0, buf1, sem_g, sem_w):
        wid = lax.axis_index("s") * NC + lax.axis_index("c")
        base = wid * b_per_w
        pltpu.sync_copy(h_hbm.at[wid], hidx)
        pltpu.sync_copy(t_hbm.at[wid], tidx)
        pltpu.sync_copy(r_hbm.at[wid], ridx)

        bufs = (buf0, buf1)
        writes = [None, None]
        step = 0
        for idx_v, table, out in ((hidx, e2_hbm, o_h),
                                  (tidx, e2_hbm, o_t),
                                  (ridx, r2_hbm, o_r)):
            for h in range(CH // 2):
                b = step % 2
                if writes[b] is not None:
                    writes[b].wait()
                g0 = pltpu.async_copy(table.at[idx_v.at[2 * h]],
                                      bufs[b].at[pl.ds(0, IW)], sem_g)
                g1 = pltpu.async_copy(table.at[idx_v.at[2 * h + 1]],
                                      bufs[b].at[pl.ds(IW, IW)], sem_g)
                g0.wait()
                g1.wait()
                writes[b] = pltpu.async_copy(
                    bufs[b], out.at[pl.ds(base + h * 2 * IW, 2 * IW)], sem_w)
                step += 1
        writes[0].wait()
        writes[1].wait()

    return k(heads, rels, tails, e2, r2)


def _tc_loss(g_h, g_t, g_r, batch):
    w = 2048
    g = batch // w

    def body(gh_p, gt_p, gr_p, gh_n, gt_n, gr_n, out):
        def mag(gh, gt, gr):
            hre, him = gh[:, :DIM], gh[:, DIM:]
            tre, tim = gt[:, :DIM], gt[:, DIM:]
            c, s = jnp.cos(gr[:, :DIM]), jnp.sin(gr[:, DIM:])
            dre = hre * c - him * s - tre
            dim = hre * s + him * c - tim
            return jnp.sum(jnp.sqrt(dre * dre + dim * dim + 1e-9), axis=-1)

        ms = jnp.maximum(MARGIN - mag(gh_n[...], gt_n[...], gr_n[...])
                         + mag(gh_p[...], gt_p[...], gr_p[...]), 0.0)
        i = pl.program_id(0)

        @pl.when(i == 0)
        def _():
            out[...] = jnp.zeros((1, 1), jnp.float32)

        out[...] += jnp.sum(ms).reshape(1, 1)

        @pl.when(i == g - 1)
        def _():
            out[...] = out[...] / batch

    pos_spec = pl.BlockSpec((w, 2 * DIM), lambda i: (i, 0))
    neg_spec = pl.BlockSpec((w, 2 * DIM), lambda i: (i + g, 0))
    out = pl.pallas_call(
        body,
        grid=(g,),
        in_specs=[pos_spec] * 3 + [neg_spec] * 3,
        out_specs=pl.BlockSpec((1, 1), lambda i: (0, 0)),
        out_shape=jax.ShapeDtypeStruct((1, 1), jnp.float32),
    )(g_h, g_t, g_r, g_h, g_t, g_r)
    return out[0, 0]


def kernel(positive_triples, negative_triples, entity_re, entity_im,
           relation_phase):
    batch = positive_triples.shape[0]
    total = 2 * batch
    pt = positive_triples.astype(jnp.int32)
    nt = negative_triples.astype(jnp.int32)
    heads = jnp.concatenate([pt[:, 0], nt[:, 0]]).reshape(NW, CH, IW)
    rels = jnp.concatenate([pt[:, 1], nt[:, 1]]).reshape(NW, CH, IW)
    tails = jnp.concatenate([pt[:, 2], nt[:, 2]]).reshape(NW, CH, IW)
    e2, r2 = _tc_prep(entity_re, entity_im, relation_phase)
    g_h, g_t, g_r = _sc_gather(heads, rels, tails, e2, r2, total)
    return _tc_loss(g_h, g_t, g_r, batch)


# speedup vs baseline: 1.2092x; 1.1588x over previous
import functools

import jax
import jax.numpy as jnp
from jax import lax
from jax.experimental import pallas as pl
from jax.experimental.pallas import tpu as pltpu
from jax.experimental.pallas import tpu_sc as plsc

DIM = 64
MARGIN = 6.0
NC, NS = 2, 16
NW = NC * NS
IW = 128
CH = 8


def _sc_gather(heads, rels, tails, e2, p2, total):
    b_per_w = CH * IW
    mesh = plsc.VectorSubcoreMesh(core_axis_name="c", subcore_axis_name="s")
    row_t = jax.ShapeDtypeStruct((total, 2 * DIM), jnp.float32)

    @functools.partial(
        pl.kernel, mesh=mesh,
        out_type=[row_t] * 3,
        scratch_types=[pltpu.VMEM((CH, IW), jnp.int32)] * 3
        + [pltpu.VMEM((2 * IW, 2 * DIM), jnp.float32)] * 2
        + [pltpu.SemaphoreType.DMA] * 2,
    )
    def k(h_hbm, r_hbm, t_hbm, e2_hbm, p2_hbm,
          o_h, o_t, o_r,
          hidx, ridx, tidx, buf0, buf1, sem_g, sem_w):
        wid = lax.axis_index("s") * NC + lax.axis_index("c")
        base = wid * b_per_w
        pltpu.sync_copy(h_hbm.at[wid], hidx)
        pltpu.sync_copy(t_hbm.at[wid], tidx)
        pltpu.sync_copy(r_hbm.at[wid], ridx)

        bufs = (buf0, buf1)
        writes = [None, None]
        step = 0
        for idx_v, table, out in ((hidx, e2_hbm, o_h),
                                  (tidx, e2_hbm, o_t),
                                  (ridx, p2_hbm, o_r)):
            for h in range(CH // 2):
                b = step % 2
                if writes[b] is not None:
                    writes[b].wait()
                g0 = pltpu.async_copy(table.at[idx_v.at[2 * h]],
                                      bufs[b].at[pl.ds(0, IW)], sem_g)
                g1 = pltpu.async_copy(table.at[idx_v.at[2 * h + 1]],
                                      bufs[b].at[pl.ds(IW, IW)], sem_g)
                g0.wait()
                g1.wait()
                writes[b] = pltpu.async_copy(
                    bufs[b], out.at[pl.ds(base + h * 2 * IW, 2 * IW)], sem_w)
                step += 1
        writes[0].wait()
        writes[1].wait()

    return k(heads, rels, tails, e2, p2)


def _tc_loss(g_h, g_t, g_r, batch):
    total = 2 * batch
    w = 1024
    g = total // w

    def body(gh, gt, gr, out, mags):
        hre, him = gh[:, :DIM], gh[:, DIM:]
        tre, tim = gt[:, :DIM], gt[:, DIM:]
        c, s = jnp.cos(gr[:, :DIM]), jnp.sin(gr[:, DIM:])
        dre = hre * c - him * s - tre
        dim = hre * s + him * c - tim
        i = pl.program_id(0)
        mags[pl.ds(i * w, w)] = jnp.sum(
            jnp.sqrt(dre * dre + dim * dim + 1e-9), axis=-1)

        @pl.when(i == g - 1)
        def _():
            mp = mags[pl.ds(0, batch)]
            mn = mags[pl.ds(batch, batch)]
            ms = jnp.maximum(MARGIN - mn + mp, 0.0)
            out[...] = (jnp.sum(ms) / batch).reshape(1, 1)

    spec = pl.BlockSpec((w, 2 * DIM), lambda i: (i, 0))
    out = pl.pallas_call(
        body,
        grid=(g,),
        in_specs=[spec] * 3,
        out_specs=pl.BlockSpec((1, 1), lambda i: (0, 0)),
        out_shape=jax.ShapeDtypeStruct((1, 1), jnp.float32),
        scratch_shapes=[pltpu.VMEM((total,), jnp.float32)],
    )(g_h, g_t, g_r)
    return out[0, 0]


def kernel(positive_triples, negative_triples, entity_re, entity_im,
           relation_phase):
    batch = positive_triples.shape[0]
    total = 2 * batch
    ch = total // (NW * IW)
    assert ch == CH
    pt = positive_triples.astype(jnp.int32)
    nt = negative_triples.astype(jnp.int32)
    heads = jnp.concatenate([pt[:, 0], nt[:, 0]]).reshape(NW, CH, IW)
    rels = jnp.concatenate([pt[:, 1], nt[:, 1]]).reshape(NW, CH, IW)
    tails = jnp.concatenate([pt[:, 2], nt[:, 2]]).reshape(NW, CH, IW)
    e2 = jnp.concatenate([entity_re, entity_im], axis=1)
    p2 = jnp.concatenate([relation_phase, relation_phase], axis=1)
    g_h, g_t, g_r = _sc_gather(heads, rels, tails, e2, p2, total)
    return _tc_loss(g_h, g_t, g_r, batch)


# speedup vs baseline: 1.4386x; 1.1897x over previous
import functools

import jax
import jax.numpy as jnp
from jax import lax
from jax.experimental import pallas as pl
from jax.experimental.pallas import tpu as pltpu
from jax.experimental.pallas import tpu_sc as plsc

DIM = 64
MARGIN = 6.0
NC, NS = 2, 16
NW = NC * NS
IW = 128
CH = 8
PB = 1024
HW = 1024


def _tc_pack(re_t, im_t, ph_t):
    n = re_t.shape[1]

    def body(re_ref, im_ref, ph_ref, e2_ref, p2_ref):
        r = re_ref[...].T
        i_ = im_ref[...].T
        p = ph_ref[...].T
        e2_ref[...] = jnp.concatenate([r, i_], axis=1)
        p2_ref[...] = jnp.concatenate([p, p], axis=1)

    in_spec = pl.BlockSpec((DIM, PB), lambda i: (0, i))
    out_spec = pl.BlockSpec((PB, 2 * DIM), lambda i: (i, 0))
    t = jax.ShapeDtypeStruct((n, 2 * DIM), jnp.float32)
    return pl.pallas_call(
        body,
        grid=(pl.cdiv(n, PB),),
        in_specs=[in_spec] * 3,
        out_specs=[out_spec] * 2,
        out_shape=[t, t],
        compiler_params=pltpu.CompilerParams(
            dimension_semantics=("arbitrary",)),
    )(re_t, im_t, ph_t)


def _sc_gather(heads, rels, tails, e2, p2, total):
    b_per_w = CH * IW
    mesh = plsc.VectorSubcoreMesh(core_axis_name="c", subcore_axis_name="s")
    row_t = jax.ShapeDtypeStruct((total, 2 * DIM), jnp.float32)

    @functools.partial(
        pl.kernel, mesh=mesh,
        out_type=[row_t] * 3,
        scratch_types=[pltpu.VMEM((CH, IW), jnp.int32)] * 3
        + [pltpu.VMEM((2 * IW, 2 * DIM), jnp.float32)] * 2
        + [pltpu.SemaphoreType.DMA] * 2,
    )
    def k(h_hbm, r_hbm, t_hbm, e2_hbm, p2_hbm,
          o_h, o_t, o_r,
          hidx, ridx, tidx, buf0, buf1, sem_g, sem_w):
        wid = lax.axis_index("s") * NC + lax.axis_index("c")
        base = wid * b_per_w
        pltpu.sync_copy(h_hbm.at[wid], hidx)
        pltpu.sync_copy(t_hbm.at[wid], tidx)
        pltpu.sync_copy(r_hbm.at[wid], ridx)

        bufs = (buf0, buf1)
        writes = [None, None]
        step = 0
        for idx_v, table, out in ((hidx, e2_hbm, o_h),
                                  (tidx, e2_hbm, o_t),
                                  (ridx, p2_hbm, o_r)):
            for h in range(CH // 2):
                b = step % 2
                if writes[b] is not None:
                    writes[b].wait()
                g0 = pltpu.async_copy(table.at[idx_v.at[2 * h]],
                                      bufs[b].at[pl.ds(0, IW)], sem_g)
                g1 = pltpu.async_copy(table.at[idx_v.at[2 * h + 1]],
                                      bufs[b].at[pl.ds(IW, IW)], sem_g)
                g0.wait()
                g1.wait()
                writes[b] = pltpu.async_copy(
                    bufs[b], out.at[pl.ds(base + h * 2 * IW, 2 * IW)], sem_w)
                step += 1
        writes[0].wait()
        writes[1].wait()

    return k(heads, rels, tails, e2, p2)


def _tc_loss(g_h, g_t, g_r, batch):
    g = batch // HW

    def body(gh, gt, gr, out):
        ones = jnp.ones((DIM, 2 * DIM), jnp.float32)

        def mag(gh_v, gt_v, gr_v):
            hre, him = gh_v[:, :DIM], gh_v[:, DIM:]
            tre, tim = gt_v[:, :DIM], gt_v[:, DIM:]
            c, s = jnp.cos(gr_v[:, :DIM]), jnp.sin(gr_v[:, DIM:])
            dre = hre * c - him * s - tre
            dim = hre * s + him * c - tim
            sq = jnp.sqrt(dre * dre + dim * dim + 1e-9)
            return jax.lax.dot_general(
                sq, ones, (((1,), (0,)), ((), ())),
                precision=jax.lax.Precision.HIGHEST,
                preferred_element_type=jnp.float32)

        gh_v, gt_v, gr_v = gh[...], gt[...], gr[...]
        mp = mag(gh_v[:HW], gt_v[:HW], gr_v[:HW])
        mn = mag(gh_v[HW:], gt_v[HW:], gr_v[HW:])
        ms = jnp.maximum(MARGIN - mn + mp, 0.0)
        i = pl.program_id(0)

        @pl.when(i == 0)
        def _():
            out[...] = jnp.zeros((1, 1), jnp.float32)

        out[...] += (jnp.sum(ms) / (2 * DIM)).reshape(1, 1)

        @pl.when(i == g - 1)
        def _():
            out[...] = out[...] / batch

    spec = pl.BlockSpec((2 * HW, 2 * DIM), lambda i: (i, 0))
    out = pl.pallas_call(
        body,
        grid=(g,),
        in_specs=[spec] * 3,
        out_specs=pl.BlockSpec((1, 1), lambda i: (0, 0)),
        out_shape=jax.ShapeDtypeStruct((1, 1), jnp.float32),
    )(g_h, g_t, g_r)
    return out[0, 0]


def kernel(positive_triples, negative_triples, entity_re, entity_im,
           relation_phase):
    batch = positive_triples.shape[0]
    total = 2 * batch
    nchunk = batch // HW
    pt = positive_triples.astype(jnp.int32)
    nt = negative_triples.astype(jnp.int32)

    def order(col_p, col_n):
        mixed = jnp.concatenate([col_p.reshape(nchunk, HW),
                                 col_n.reshape(nchunk, HW)], axis=1)
        return mixed.reshape(NW, CH, IW)

    heads = order(pt[:, 0], nt[:, 0])
    rels = order(pt[:, 1], nt[:, 1])
    tails = order(pt[:, 2], nt[:, 2])
    e2, p2 = _tc_pack(entity_re.T, entity_im.T, relation_phase.T)
    g_h, g_t, g_r = _sc_gather(heads, rels, tails, e2, p2, total)
    return _tc_loss(g_h, g_t, g_r, batch)


# speedup vs baseline: 1.8972x; 1.3188x over previous
import functools

import jax
import jax.numpy as jnp
from jax import lax
from jax.experimental import pallas as pl
from jax.experimental.pallas import tpu as pltpu
from jax.experimental.pallas import tpu_sc as plsc

DIM = 64
MARGIN = 6.0
NC, NS = 2, 16
NW = NC * NS
IW = 128
CH = 8
PB = 2048
HW = 1024


def _tc_pack_e2(re_t, im_t):
    n = re_t.shape[1]

    def body(re_ref, im_ref, e2_ref):
        e2_ref[...] = jnp.concatenate([re_ref[...].T, im_ref[...].T], axis=1)

    return pl.pallas_call(
        body,
        grid=(pl.cdiv(n, PB),),
        in_specs=[pl.BlockSpec((DIM, PB), lambda i: (0, i))] * 2,
        out_specs=pl.BlockSpec((PB, 2 * DIM), lambda i: (i, 0)),
        out_shape=jax.ShapeDtypeStruct((n, 2 * DIM), jnp.float32),
    )(re_t, im_t)


def _tc_pack_p2(ph_t):
    n = ph_t.shape[1]

    def body(ph_ref, p2_ref):
        p = ph_ref[...].T
        p2_ref[...] = jnp.concatenate([p, p], axis=1)

    return pl.pallas_call(
        body,
        grid=(pl.cdiv(n, PB),),
        in_specs=[pl.BlockSpec((DIM, PB), lambda i: (0, i))],
        out_specs=pl.BlockSpec((PB, 2 * DIM), lambda i: (i, 0)),
        out_shape=jax.ShapeDtypeStruct((n, 2 * DIM), jnp.float32),
    )(ph_t)


def _sc_gather(idx_list, table, total):
    nrole = len(idx_list)
    b_per_w = CH * IW
    mesh = plsc.VectorSubcoreMesh(core_axis_name="c", subcore_axis_name="s")
    row_t = jax.ShapeDtypeStruct((total, 2 * DIM), jnp.float32)

    @functools.partial(
        pl.kernel, mesh=mesh,
        out_type=[row_t] * nrole,
        scratch_types=[pltpu.VMEM((CH, IW), jnp.int32)] * nrole
        + [pltpu.VMEM((2 * IW, 2 * DIM), jnp.float32)] * 2
        + [pltpu.SemaphoreType.DMA] * 2,
    )
    def k(*refs):
        idx_hbm = refs[:nrole]
        table_hbm = refs[nrole]
        outs = refs[nrole + 1:2 * nrole + 1]
        idx_v = refs[2 * nrole + 1:3 * nrole + 1]
        buf0, buf1, sem_g, sem_w = refs[3 * nrole + 1:]
        wid = lax.axis_index("s") * NC + lax.axis_index("c")
        base = wid * b_per_w
        for r in range(nrole):
            pltpu.sync_copy(idx_hbm[r].at[wid], idx_v[r])

        bufs = (buf0, buf1)
        writes = [None, None]
        step = 0
        for r in range(nrole):
            for h in range(CH // 2):
                b = step % 2
                if writes[b] is not None:
                    writes[b].wait()
                g0 = pltpu.async_copy(table_hbm.at[idx_v[r].at[2 * h]],
                                      bufs[b].at[pl.ds(0, IW)], sem_g)
                g1 = pltpu.async_copy(table_hbm.at[idx_v[r].at[2 * h + 1]],
                                      bufs[b].at[pl.ds(IW, IW)], sem_g)
                g0.wait()
                g1.wait()
                writes[b] = pltpu.async_copy(
                    bufs[b], outs[r].at[pl.ds(base + h * 2 * IW, 2 * IW)],
                    sem_w)
                step += 1
        writes[0].wait()
        writes[1].wait()

    return k(*idx_list, table)


def _tc_loss(g_h, g_t, g_r, batch):
    g = batch // HW

    sin_c = (0.9999999378197463, -0.16666621108235025, 0.008332791502704946,
             -0.00019817630987702638, 2.70883115859738e-06,
             -2.0698134650665168e-08)
    cos_c = (0.9999992107795053, -0.4999942133837966, 0.041659777806388416,
             -0.0013858789919373926, 2.4202941365944475e-05,
             -2.1972963820671154e-07)

    def body(gh, gt, gr, out):
        ones = jnp.ones((2 * DIM, 2 * DIM), jnp.float32)
        mask = lax.broadcasted_iota(jnp.int32, (2 * HW, 2 * DIM), 1) < DIM
        mrow = lax.broadcasted_iota(jnp.int32, (1, 2 * DIM), 1) < DIM
        coef = [jnp.where(mrow, c, s).astype(jnp.float32)
                for c, s in zip(cos_c, sin_c)]

        def swap(x):
            return jnp.roll(x, DIM, axis=1)

        a = gh[...]
        t = gt[...]
        r = gr[...]
        y = r * r
        p = coef[5]
        for k in (4, 3, 2, 1, 0):
            p = p * y + coef[k]
        cs = jnp.where(mask, p, p * r)
        u = a * cs
        v = a * swap(cs)
        dre2 = u - swap(u)
        dim2 = v + swap(v)
        rot = jnp.where(mask, dre2, dim2)
        diff = rot - t
        sq = diff * diff
        val = jnp.sqrt(sq + swap(sq) + 1e-9)
        mag = jax.lax.dot_general(
            val, ones, (((1,), (0,)), ((), ())),
            precision=jax.lax.Precision.HIGHEST,
            preferred_element_type=jnp.float32)
        ms = jnp.maximum(MARGIN + 0.5 * (mag[:HW] - mag[HW:]), 0.0)
        i = pl.program_id(0)

        @pl.when(i == 0)
        def _():
            out[...] = jnp.zeros((1, 1), jnp.float32)

        out[...] += (jnp.sum(ms) / (2 * DIM)).reshape(1, 1)

        @pl.when(i == g - 1)
        def _():
            out[...] = out[...] / batch

    spec = pl.BlockSpec((2 * HW, 2 * DIM), lambda i: (i, 0))
    out = pl.pallas_call(
        body,
        grid=(g,),
        in_specs=[spec] * 3,
        out_specs=pl.BlockSpec((1, 1), lambda i: (0, 0)),
        out_shape=jax.ShapeDtypeStruct((1, 1), jnp.float32),
    )(g_h, g_t, g_r)
    return out[0, 0]


def kernel(positive_triples, negative_triples, entity_re, entity_im,
           relation_phase):
    batch = positive_triples.shape[0]
    total = 2 * batch
    nchunk = batch // HW
    pt = positive_triples.astype(jnp.int32)
    nt = negative_triples.astype(jnp.int32)

    def order(col_p, col_n):
        mixed = jnp.concatenate([col_p.reshape(nchunk, HW),
                                 col_n.reshape(nchunk, HW)], axis=1)
        return mixed.reshape(NW, CH, IW)

    heads = order(pt[:, 0], nt[:, 0])
    rels = order(pt[:, 1], nt[:, 1])
    tails = order(pt[:, 2], nt[:, 2])
    e2 = _tc_pack_e2(entity_re.T, entity_im.T)
    g_h, g_t = _sc_gather([heads, tails], e2, total)
    p2 = _tc_pack_p2(relation_phase.T)
    (g_r,) = _sc_gather([rels], p2, total)
    return _tc_loss(g_h, g_t, g_r, batch)


# speedup vs baseline: 2.2037x; 1.1616x over previous
import functools

import jax
import jax.numpy as jnp
from jax import lax
from jax.experimental import pallas as pl
from jax.experimental.pallas import tpu as pltpu
from jax.experimental.pallas import tpu_sc as plsc

DIM = 64
MARGIN = 6.0
NC, NS = 2, 16
NW = NC * NS
IW = 128
CH = 8
PB = 4096
HW = 1024


def _tc_pack_e2(re_t, im_t):
    n = re_t.shape[1]

    def body(re_ref, im_ref, e2_ref):
        e2_ref[...] = jnp.concatenate([re_ref[...].T, im_ref[...].T], axis=1)

    return pl.pallas_call(
        body,
        grid=(pl.cdiv(n, PB),),
        in_specs=[pl.BlockSpec((DIM, PB), lambda i: (0, i))] * 2,
        out_specs=pl.BlockSpec((PB, 2 * DIM), lambda i: (i, 0)),
        out_shape=jax.ShapeDtypeStruct((n, 2 * DIM), jnp.float32),
    )(re_t, im_t)


def _tc_pack_p2(ph_t):
    n = ph_t.shape[1]

    def body(ph_ref, p2_ref):
        p = ph_ref[...].T
        p2_ref[...] = jnp.concatenate([p, p], axis=1)

    return pl.pallas_call(
        body,
        grid=(pl.cdiv(n, PB),),
        in_specs=[pl.BlockSpec((DIM, PB), lambda i: (0, i))],
        out_specs=pl.BlockSpec((PB, 2 * DIM), lambda i: (i, 0)),
        out_shape=jax.ShapeDtypeStruct((n, 2 * DIM), jnp.float32),
    )(ph_t)


def _sc_gather(idx_list, table, total):
    nrole = len(idx_list)
    b_per_w = CH * IW
    mesh = plsc.VectorSubcoreMesh(core_axis_name="c", subcore_axis_name="s")
    row_t = jax.ShapeDtypeStruct((total, 2 * DIM), jnp.float32)

    @functools.partial(
        pl.kernel, mesh=mesh,
        out_type=[row_t] * nrole,
        scratch_types=[pltpu.VMEM((CH, IW), jnp.int32)] * nrole
        + [pltpu.VMEM((2 * IW, 2 * DIM), jnp.float32)] * 2
        + [pltpu.SemaphoreType.DMA] * 2,
    )
    def k(*refs):
        idx_hbm = refs[:nrole]
        table_hbm = refs[nrole]
        outs = refs[nrole + 1:2 * nrole + 1]
        idx_v = refs[2 * nrole + 1:3 * nrole + 1]
        buf0, buf1, sem_g, sem_w = refs[3 * nrole + 1:]
        wid = lax.axis_index("s") * NC + lax.axis_index("c")
        base = wid * b_per_w
        for r in range(nrole):
            pltpu.sync_copy(idx_hbm[r].at[wid], idx_v[r])

        bufs = (buf0, buf1)
        writes = [None, None]
        step = 0
        for r in range(nrole):
            for h in range(CH // 2):
                b = step % 2
                if writes[b] is not None:
                    writes[b].wait()
                g0 = pltpu.async_copy(table_hbm.at[idx_v[r].at[2 * h]],
                                      bufs[b].at[pl.ds(0, IW)], sem_g)
                g1 = pltpu.async_copy(table_hbm.at[idx_v[r].at[2 * h + 1]],
                                      bufs[b].at[pl.ds(IW, IW)], sem_g)
                g0.wait()
                g1.wait()
                writes[b] = pltpu.async_copy(
                    bufs[b], outs[r].at[pl.ds(base + h * 2 * IW, 2 * IW)],
                    sem_w)
                step += 1
        writes[0].wait()
        writes[1].wait()

    return k(*idx_list, table)


def _tc_loss(g_h, g_t, g_r, batch):
    g = batch // HW

    sin_c = (0.9999999378197463, -0.16666621108235025, 0.008332791502704946,
             -0.00019817630987702638, 2.70883115859738e-06,
             -2.0698134650665168e-08)
    cos_c = (0.9999992107795053, -0.4999942133837966, 0.041659777806388416,
             -0.0013858789919373926, 2.4202941365944475e-05,
             -2.1972963820671154e-07)

    def body(gh, gt, gr, out):
        ones = jnp.ones((2 * DIM, 2 * DIM), jnp.float32)
        mask = lax.broadcasted_iota(jnp.int32, (2 * HW, 2 * DIM), 1) < DIM
        mrow = lax.broadcasted_iota(jnp.int32, (1, 2 * DIM), 1) < DIM
        coef = [jnp.where(mrow, c, s).astype(jnp.float32)
                for c, s in zip(cos_c, sin_c)]

        def swap(x):
            return jnp.roll(x, DIM, axis=1)

        a = gh[...]
        t = gt[...]
        r = gr[...]
        y = r * r
        p = coef[5]
        for k in (4, 3, 2, 1, 0):
            p = p * y + coef[k]
        cs = jnp.where(mask, p, p * r)
        u = a * cs
        v = a * swap(cs)
        dre2 = u - swap(u)
        dim2 = v + swap(v)
        rot = jnp.where(mask, dre2, dim2)
        diff = rot - t
        sq = diff * diff
        val = jnp.sqrt(sq + swap(sq) + 1e-9)
        mag = jax.lax.dot_general(
            val, ones, (((1,), (0,)), ((), ())),
            preferred_element_type=jnp.float32)
        ms = jnp.maximum(MARGIN + 0.5 * (mag[:HW] - mag[HW:]), 0.0)
        i = pl.program_id(0)

        @pl.when(i == 0)
        def _():
            out[...] = jnp.zeros((1, 1), jnp.float32)

        out[...] += (jnp.sum(ms) / (2 * DIM)).reshape(1, 1)

        @pl.when(i == g - 1)
        def _():
            out[...] = out[...] / batch

    spec = pl.BlockSpec((2 * HW, 2 * DIM), lambda i: (i, 0))
    out = pl.pallas_call(
        body,
        grid=(g,),
        in_specs=[spec] * 3,
        out_specs=pl.BlockSpec((1, 1), lambda i: (0, 0)),
        out_shape=jax.ShapeDtypeStruct((1, 1), jnp.float32),
    )(g_h, g_t, g_r)
    return out[0, 0]


def kernel(positive_triples, negative_triples, entity_re, entity_im,
           relation_phase):
    batch = positive_triples.shape[0]
    total = 2 * batch
    nchunk = batch // HW
    pt = positive_triples.astype(jnp.int32)
    nt = negative_triples.astype(jnp.int32)

    def order(col_p, col_n):
        mixed = jnp.concatenate([col_p.reshape(nchunk, HW),
                                 col_n.reshape(nchunk, HW)], axis=1)
        return mixed.reshape(NW, CH, IW)

    heads = order(pt[:, 0], nt[:, 0])
    rels = order(pt[:, 1], nt[:, 1])
    tails = order(pt[:, 2], nt[:, 2])
    e2 = _tc_pack_e2(entity_re.T, entity_im.T)
    g_h, g_t = _sc_gather([heads, tails], e2, total)
    p2 = _tc_pack_p2(relation_phase.T)
    (g_r,) = _sc_gather([rels], p2, total)
    return _tc_loss(g_h, g_t, g_r, batch)
